# baseline (speedup 1.0000x reference)
import functools

import jax
import jax.numpy as jnp
from jax import lax
from jax.experimental import pallas as pl
from jax.experimental.pallas import tpu as pltpu
from jax.experimental.pallas import tpu_sc as plsc

NC = 2
NS = 16
NW = NC * NS
LANES = 16


def _make_kernel(B, K):
    assert B % NW == 0
    b_per_w = B // NW
    C = min(128, b_per_w)
    n_chunks = b_per_w // C
    k_regs = K // LANES
    n_groups = C // LANES

    mesh = plsc.VectorSubcoreMesh(core_axis_name="c", subcore_axis_name="s")

    @functools.partial(
        pl.kernel,
        out_type=jax.ShapeDtypeStruct((B,), jnp.float32),
        mesh=mesh,
        compiler_params=pltpu.CompilerParams(needs_layout_passes=False),
        scratch_types=[
            pltpu.VMEM((b_per_w,), jnp.int32),
            pltpu.VMEM((b_per_w,), jnp.int32),
            pltpu.VMEM((C, K), jnp.float32),
            pltpu.VMEM((C, K), jnp.float32),
            pltpu.VMEM((C,), jnp.float32),
            pltpu.VMEM((C,), jnp.float32),
            pltpu.VMEM((LANES,), jnp.float32),
            pltpu.VMEM((b_per_w,), jnp.float32),
            pltpu.SemaphoreType.DMA,
        ],
    )
    def cf_kernel(uidx_hbm, iidx_hbm, uemb_hbm, iemb_hbm, ubias_hbm,
                  ibias_hbm, gbias_hbm, out_hbm,
                  uidx_v, iidx_v, urows_v, irows_v, ub_v, ib_v, gb_v,
                  out_v, sem):
        wid = lax.axis_index("s") * NC + lax.axis_index("c")
        base = wid * b_per_w
        pltpu.sync_copy(uidx_hbm.at[pl.ds(base, b_per_w)], uidx_v)
        pltpu.sync_copy(iidx_hbm.at[pl.ds(base, b_per_w)], iidx_v)
        pltpu.sync_copy(gbias_hbm, gb_v.at[pl.ds(0, 1)])
        gb = gb_v[pl.ds(0, LANES)][0]
        lane = lax.iota(jnp.int32, LANES)

        for c in range(n_chunks):
            u_idx = uidx_v.at[pl.ds(c * C, C)]
            i_idx = iidx_v.at[pl.ds(c * C, C)]
            cps = [
                pltpu.async_copy(uemb_hbm.at[u_idx], urows_v, sem),
                pltpu.async_copy(iemb_hbm.at[i_idx], irows_v, sem),
                pltpu.async_copy(ubias_hbm.at[u_idx], ub_v, sem),
                pltpu.async_copy(ibias_hbm.at[i_idx], ib_v, sem),
            ]
            for cp in cps:
                cp.wait()

            def group_body(g, _):
                goff = g * LANES
                dots = jnp.zeros((LANES,), jnp.float32)
                for e in range(LANES):
                    row = goff + e
                    acc = (urows_v[row, pl.ds(0, LANES)]
                           * irows_v[row, pl.ds(0, LANES)])
                    for j in range(1, k_regs):
                        acc = acc + (urows_v[row, pl.ds(j * LANES, LANES)]
                                     * irows_v[row, pl.ds(j * LANES, LANES)])
                    dots = jnp.where(lane == e, jnp.sum(acc), dots)
                out_v[pl.ds(c * C + goff, LANES)] = (
                    dots + ub_v[pl.ds(goff, LANES)] + ib_v[pl.ds(goff, LANES)]
                    + gb)
                return 0

            lax.fori_loop(0, n_groups, group_body, 0)

        pltpu.sync_copy(out_v, out_hbm.at[pl.ds(base, b_per_w)])

    return cf_kernel


def kernel(user_input, item_input, user_emb, item_emb, user_bias, item_bias,
           global_bias):
    B = user_input.shape[0]
    K = user_emb.shape[1]
    k = _make_kernel(B, K)
    return k(user_input.astype(jnp.int32), item_input.astype(jnp.int32),
             user_emb, item_emb,
             user_bias.reshape(-1), item_bias.reshape(-1), global_bias)


# speedup vs baseline: 1.3248x; 1.3248x over previous
import functools

import jax
import jax.numpy as jnp
from jax import lax
from jax.experimental import pallas as pl
from jax.experimental.pallas import tpu as pltpu
from jax.experimental.pallas import tpu_sc as plsc

NC = 2
NS = 16
NW = NC * NS
LANES = 16


def _tree_sum(vals):
    while len(vals) > 1:
        nxt = [a + b for a, b in zip(vals[::2], vals[1::2])]
        if len(vals) % 2:
            nxt.append(vals[-1])
        vals = nxt
    return vals[0]


def _make_kernel(B, K):
    assert B % NW == 0
    b_per_w = B // NW
    C = min(128, b_per_w)
    n_chunks = b_per_w // C
    k_regs = K // LANES
    n_groups = C // LANES

    mesh = plsc.VectorSubcoreMesh(core_axis_name="c", subcore_axis_name="s")

    @functools.partial(
        pl.kernel,
        out_type=jax.ShapeDtypeStruct((B,), jnp.float32),
        mesh=mesh,
        compiler_params=pltpu.CompilerParams(needs_layout_passes=False),
        scratch_types=[
            pltpu.VMEM((b_per_w,), jnp.int32),
            pltpu.VMEM((b_per_w,), jnp.int32),
            pltpu.VMEM((C, K), jnp.float32),
            pltpu.VMEM((C, K), jnp.float32),
            pltpu.VMEM((C, K), jnp.float32),
            pltpu.VMEM((C, K), jnp.float32),
            pltpu.VMEM((C,), jnp.float32),
            pltpu.VMEM((C,), jnp.float32),
            pltpu.VMEM((C,), jnp.float32),
            pltpu.VMEM((C,), jnp.float32),
            pltpu.VMEM((LANES,), jnp.float32),
            pltpu.VMEM((LANES, LANES + 1), jnp.float32),
            pltpu.VMEM((b_per_w,), jnp.float32),
            pltpu.SemaphoreType.DMA,
            pltpu.SemaphoreType.DMA,
        ],
    )
    def cf_kernel(uidx_hbm, iidx_hbm, uemb_hbm, iemb_hbm, ubias_hbm,
                  ibias_hbm, gbias_hbm, out_hbm,
                  uidx_v, iidx_v, urows0, urows1, irows0, irows1,
                  ub0, ub1, ib0, ib1, gb_v, mat_v, out_v, sem0, sem1):
        urows = (urows0, urows1)
        irows = (irows0, irows1)
        ub = (ub0, ub1)
        ib = (ib0, ib1)
        sems = (sem0, sem1)

        wid = lax.axis_index("s") * NC + lax.axis_index("c")
        base = wid * b_per_w
        pltpu.sync_copy(uidx_hbm.at[pl.ds(base, b_per_w)], uidx_v)
        pltpu.sync_copy(iidx_hbm.at[pl.ds(base, b_per_w)], iidx_v)
        pltpu.sync_copy(gbias_hbm, gb_v.at[pl.ds(0, 1)])
        gb = gb_v[pl.ds(0, LANES)][0]
        lane = lax.iota(jnp.int32, LANES)

        def issue(c, bi):
            u_idx = uidx_v.at[pl.ds(c * C, C)]
            i_idx = iidx_v.at[pl.ds(c * C, C)]
            return [
                pltpu.async_copy(uemb_hbm.at[u_idx], urows[bi], sems[bi]),
                pltpu.async_copy(iemb_hbm.at[i_idx], irows[bi], sems[bi]),
                pltpu.async_copy(ubias_hbm.at[u_idx], ub[bi], sems[bi]),
                pltpu.async_copy(ibias_hbm.at[i_idx], ib[bi], sems[bi]),
            ]

        pending = issue(0, 0)
        for c in range(n_chunks):
            bi = c % 2
            nxt = issue(c + 1, 1 - bi) if c + 1 < n_chunks else []
            for cp in pending:
                cp.wait()
            pending = nxt

            uro, iro, ubo, ibo = urows[bi], irows[bi], ub[bi], ib[bi]

            def group_body(g, _):
                goff = g * LANES
                for e in range(LANES):
                    row = goff + e
                    prods = [uro[row, pl.ds(j * LANES, LANES)]
                             * iro[row, pl.ds(j * LANES, LANES)]
                             for j in range(k_regs)]
                    mat_v[e, pl.ds(0, LANES)] = _tree_sum(prods)
                cols = [plsc.load_gather(
                            mat_v, [lane, jnp.full((LANES,), j, jnp.int32)])
                        for j in range(LANES)]
                dots = _tree_sum(cols)
                out_v[pl.ds(c * C + goff, LANES)] = (
                    dots + ubo[pl.ds(goff, LANES)] + ibo[pl.ds(goff, LANES)]
                    + gb)
                return 0

            lax.fori_loop(0, n_groups, group_body, 0)

        pltpu.sync_copy(out_v, out_hbm.at[pl.ds(base, b_per_w)])

    return cf_kernel


def kernel(user_input, item_input, user_emb, item_emb, user_bias, item_bias,
           global_bias):
    B = user_input.shape[0]
    K = user_emb.shape[1]
    k = _make_kernel(B, K)
    return k(user_input.astype(jnp.int32), item_input.astype(jnp.int32),
             user_emb, item_emb,
             user_bias.reshape(-1), item_bias.reshape(-1), global_bias)
